# baseline (speedup 1.0000x reference)
import functools

import jax
import jax.numpy as jnp
from jax import lax
from jax.experimental import pallas as pl
from jax.experimental.pallas import tpu as pltpu
from jax.experimental.pallas import tpu_sc as plsc

_NW = 32
_NC = 2
_LANES = 16

_F = 26
_D = 32
_NB = 1
_GROW = 32
_LIN_OFF = 32


def _ffm_body(rows_hbm, lin_hbm, ffm_hbm, out_hbm, idxv, a_v, lin_v, out_v,
              acc_v, sem_i, sem_a, *, bpw):
    wid = lax.axis_index("s") * _NC + lax.axis_index("c")
    ngrp = bpw // _NB
    base = wid * ngrp

    def issue_gathers(buf):
        ids = idxv.at[buf, pl.ds(0, _GROW)]
        for j in range(_F):
            pltpu.async_copy(ffm_hbm.at[j].at[ids], a_v.at[buf, j],
                             sem_a.at[buf])
        pltpu.async_copy(
            lin_hbm.at[idxv.at[buf, pl.ds(_LIN_OFF, 2 * _LANES)]],
            lin_v.at[buf],
            sem_a.at[buf],
        )

    def wait_gathers(buf):
        ids = idxv.at[buf, pl.ds(0, _GROW)]
        for j in range(_F):
            pltpu.make_async_copy(ffm_hbm.at[j].at[ids], a_v.at[buf, j],
                                  sem_a.at[buf]).wait()
        pltpu.make_async_copy(
            lin_hbm.at[idxv.at[buf, pl.ds(_LIN_OFF, 2 * _LANES)]],
            lin_v.at[buf],
            sem_a.at[buf],
        ).wait()

    lane = lax.broadcasted_iota(jnp.int32, (_LANES,), 0)

    def compute(buf, s):
        l0 = lin_v[buf, pl.ds(0, _LANES)]
        l1 = lin_v[buf, pl.ds(_LANES, _LANES)]
        acc0 = l0
        acc1 = jnp.where(lane < _F - _LANES, l1, 0.0)
        for i in range(_F):
            for j in range(i + 1, _F):
                u0 = a_v[buf, j, i, pl.ds(0, _LANES)]
                v0 = a_v[buf, i, j, pl.ds(0, _LANES)]
                acc0 = acc0 + u0 * v0
                u1 = a_v[buf, j, i, pl.ds(_LANES, _LANES)]
                v1 = a_v[buf, i, j, pl.ds(_LANES, _LANES)]
                acc1 = acc1 + u1 * v1
        acc_v[pl.ds(lax.rem(s, _LANES) * _LANES, _LANES)] = acc0 + acc1

    def step(g, buf):
        nbuf = 1 - buf
        wait_gathers(buf)

        @pl.when(g + 2 < ngrp)
        def _():
            pltpu.async_copy(rows_hbm.at[base + g + 2], idxv.at[buf],
                             sem_i.at[buf])

        @pl.when(g + 1 < ngrp)
        def _():
            pltpu.make_async_copy(rows_hbm.at[base], idxv.at[nbuf],
                                  sem_i.at[nbuf]).wait()
            issue_gathers(nbuf)

        compute(buf, g)

    pltpu.sync_copy(rows_hbm.at[base], idxv.at[0])
    issue_gathers(0)
    pltpu.async_copy(rows_hbm.at[base + 1], idxv.at[1], sem_i.at[1])

    def body(t, carry):
        step(2 * t, 0)
        step(2 * t + 1, 1)

        @pl.when(lax.rem(t, 8) == 7)
        def _():
            total = jnp.zeros((_LANES,), jnp.float32)
            for l in range(_LANES):
                total = total + plsc.load_gather(acc_v, [lane * _LANES + l])
            sig = 1.0 / (1.0 + jnp.exp(-total))
            out_v[pl.ds(2 * t - 14, _LANES)] = sig

        return carry

    lax.fori_loop(0, ngrp // 2, body, jnp.int32(0))
    pltpu.sync_copy(out_v, out_hbm.at[pl.ds(wid * bpw, bpw)])


def kernel(x, offsets, lin_table, lin_bias, ffm_tables):
    b, f = x.shape
    fv, v, d = ffm_tables.shape
    assert f == _F and d == _D and b % (2 * _NW) == 0
    bpw = b // _NW

    idx = x + offsets[None, :]
    z6 = jnp.zeros((b, 6), jnp.int32)
    rows = jnp.concatenate([idx, z6, idx, z6], axis=1)

    lin_flat = lin_table[:, 0] + lin_bias[0] / _F

    mesh = plsc.VectorSubcoreMesh(core_axis_name="c", subcore_axis_name="s")
    run = pl.kernel(
        functools.partial(_ffm_body, bpw=bpw),
        out_type=jax.ShapeDtypeStruct((b,), jnp.float32),
        mesh=mesh,
        compiler_params=pltpu.CompilerParams(
            needs_layout_passes=False, use_tc_tiling_on_sc=False),
        scratch_types=[
            pltpu.VMEM((2, 64), jnp.int32),
            pltpu.VMEM((2, _F, _GROW, _D), jnp.float32),
            pltpu.VMEM((2, 2 * _LANES), jnp.float32),
            pltpu.VMEM((bpw,), jnp.float32),
            pltpu.VMEM((_LANES * _LANES,), jnp.float32),
            pltpu.SemaphoreType.DMA((2,)),
            pltpu.SemaphoreType.DMA((2,)),
        ],
    )
    return run(rows, lin_flat, ffm_tables)


# speedup vs baseline: 1.8047x; 1.8047x over previous
import functools

import jax
import jax.numpy as jnp
from jax import lax
from jax.experimental import pallas as pl
from jax.experimental.pallas import tpu as pltpu
from jax.experimental.pallas import tpu_sc as plsc

_NW = 32
_NC = 2
_LANES = 16

_F = 26
_D = 32
_W = 896
_VB = 256
_NROW = 184


def _tpose_body(x_ref, lin_ref, o_ref):
    xx = x_ref[...].reshape(_F * _D, _VB)
    xx = jnp.concatenate(
        [xx, lin_ref[...].reshape(1, _VB),
         jnp.zeros((_W - _F * _D - 1, _VB), jnp.float32)], axis=0)
    xt = jax.lax.transpose(xx, (1, 0))
    for c in range(_W // 128):
        o_ref[c] = xt[:, c * 128:(c + 1) * 128]


def _ffm_body(rows_hbm, t2_hbm, out_hbm, idxv, a_v, out_v,
              acc_v, sem_i, sem_a, *, bpw):
    wid = lax.axis_index("s") * _NC + lax.axis_index("c")
    base = wid * bpw

    def issue_gathers(buf):
        pltpu.async_copy(t2_hbm.at[idxv.at[buf]], a_v.at[buf],
                         sem_a.at[buf])

    def wait_gathers(buf):
        pltpu.make_async_copy(t2_hbm.at[idxv.at[buf]], a_v.at[buf],
                              sem_a.at[buf]).wait()

    lane = lax.broadcasted_iota(jnp.int32, (_LANES,), 0)

    def compute(buf, s):
        accs = [jnp.zeros((_LANES,), jnp.float32) for _ in range(4)]
        for i in range(_F):
            accs[i % 4] = accs[i % 4] + a_v[buf, 6 * _F + i,
                                            pl.ds(64, _LANES)]
        for i in range(_F):
            for j in range(i + 1, _F):
                jc, jo = divmod(j * _D, 128)
                ic, io = divmod(i * _D, 128)
                u0 = a_v[buf, jc * _F + i, pl.ds(jo, _LANES)]
                v0 = a_v[buf, ic * _F + j, pl.ds(io, _LANES)]
                u1 = a_v[buf, jc * _F + i, pl.ds(jo + _LANES, _LANES)]
                v1 = a_v[buf, ic * _F + j, pl.ds(io + _LANES, _LANES)]
                p = (i + j) % 2
                accs[p] = accs[p] + u0 * v0
                accs[p + 2] = accs[p + 2] + u1 * v1
        acc_v[pl.ds(lax.rem(s, _LANES) * _LANES, _LANES)] = (
            (accs[0] + accs[1]) + (accs[2] + accs[3]))

    def step(s, buf):
        nbuf = 1 - buf

        @pl.when(s + 1 < bpw)
        def _():
            pltpu.make_async_copy(rows_hbm.at[base], idxv.at[nbuf],
                                  sem_i.at[nbuf]).wait()
            issue_gathers(nbuf)

        wait_gathers(buf)

        @pl.when(s + 2 < bpw)
        def _():
            pltpu.async_copy(rows_hbm.at[base + s + 2], idxv.at[buf],
                             sem_i.at[buf])

        compute(buf, s)

    pltpu.sync_copy(rows_hbm.at[base], idxv.at[0])
    issue_gathers(0)
    pltpu.async_copy(rows_hbm.at[base + 1], idxv.at[1], sem_i.at[1])

    def body(t, carry):
        step(2 * t, 0)
        step(2 * t + 1, 1)

        @pl.when(lax.rem(t, 8) == 7)
        def _():
            total = jnp.zeros((_LANES,), jnp.float32)
            for l in range(_LANES):
                total = total + plsc.load_gather(acc_v, [lane * _LANES + l])
            sig = 1.0 / (1.0 + jnp.exp(-total))
            out_v[pl.ds(2 * t - 14, _LANES)] = sig

        return carry

    lax.fori_loop(0, bpw // 2, body, jnp.int32(0))
    pltpu.sync_copy(out_v, out_hbm.at[pl.ds(base, bpw)])


def kernel(x, offsets, lin_table, lin_bias, ffm_tables):
    b, f = x.shape
    fv, v, d = ffm_tables.shape
    assert f == _F and d == _D and b % (2 * _NW) == 0
    bpw = b // _NW

    ffm_vm = jnp.transpose(ffm_tables, (0, 2, 1))
    ngrid = (v + _VB - 1) // _VB
    lin_flat = lin_table[:, 0] + lin_bias[0] / _F
    t2 = pl.pallas_call(
        _tpose_body,
        grid=(ngrid,),
        in_specs=[pl.BlockSpec((_F, _D, _VB), lambda t: (0, 0, t)),
                  pl.BlockSpec((_VB,), lambda t: (t,))],
        out_specs=pl.BlockSpec((_W // 128, _VB, 128), lambda t: (0, t, 0)),
        out_shape=jax.ShapeDtypeStruct((_W // 128, v, 128), jnp.float32),
    )(ffm_vm, lin_flat)

    idx = x + offsets[None, :]
    coff = jnp.arange(_W // 128, dtype=jnp.int32) * v
    ffm_ids = (coff[None, :, None] + idx[:, None, :]).reshape(b, -1)
    rows = jnp.concatenate(
        [ffm_ids, jnp.zeros((b, _NROW - ffm_ids.shape[1]), jnp.int32)],
        axis=1)

    mesh = plsc.VectorSubcoreMesh(core_axis_name="c", subcore_axis_name="s")
    run = pl.kernel(
        functools.partial(_ffm_body, bpw=bpw),
        out_type=jax.ShapeDtypeStruct((b,), jnp.float32),
        mesh=mesh,
        compiler_params=pltpu.CompilerParams(
            needs_layout_passes=False, use_tc_tiling_on_sc=False),
        scratch_types=[
            pltpu.VMEM((2, _NROW), jnp.int32),
            pltpu.VMEM((2, _NROW, 128), jnp.float32),
            pltpu.VMEM((bpw,), jnp.float32),
            pltpu.VMEM((_LANES * _LANES,), jnp.float32),
            pltpu.SemaphoreType.DMA((2,)),
            pltpu.SemaphoreType.DMA((2,)),
        ],
    )
    return run(rows, t2.reshape((_W // 128) * v, 128))


# speedup vs baseline: 1.8193x; 1.0081x over previous
import functools

import jax
import jax.numpy as jnp
from jax import lax
from jax.experimental import pallas as pl
from jax.experimental.pallas import tpu as pltpu
from jax.experimental.pallas import tpu_sc as plsc

_NW = 32
_NC = 2
_LANES = 16

_F = 26
_D = 32
_W = 896
_VB = 256
_NROW = 184


def _tpose_body(x_ref, o_ref):
    xx = x_ref[...].reshape(_F * _D, _VB)
    xx = jnp.concatenate(
        [xx, jnp.zeros((_W - _F * _D, _VB), jnp.float32)], axis=0)
    xt = jax.lax.transpose(xx, (1, 0))
    for c in range(_W // 128):
        o_ref[c] = xt[:, c * 128:(c + 1) * 128]


def _ffm_body(rows_hbm, lin_hbm, t2_hbm, out_hbm, idxv, a_v, lin_v, out_v,
              acc_v, sem_i, sem_a, *, bpw):
    wid = lax.axis_index("s") * _NC + lax.axis_index("c")
    base = wid * bpw

    def issue_gathers(buf):
        pltpu.async_copy(t2_hbm.at[idxv.at[buf, pl.ds(0, _NROW)]],
                         a_v.at[buf], sem_a.at[buf])
        pltpu.async_copy(lin_hbm.at[idxv.at[buf, pl.ds(_NROW, 2 * _LANES)]],
                         lin_v.at[buf], sem_a.at[buf])

    def wait_gathers(buf):
        pltpu.make_async_copy(t2_hbm.at[idxv.at[buf, pl.ds(0, _NROW)]],
                              a_v.at[buf], sem_a.at[buf]).wait()
        pltpu.make_async_copy(
            lin_hbm.at[idxv.at[buf, pl.ds(_NROW, 2 * _LANES)]],
            lin_v.at[buf], sem_a.at[buf]).wait()

    lane = lax.broadcasted_iota(jnp.int32, (_LANES,), 0)

    def compute(buf, s):
        l0 = lin_v[buf, pl.ds(0, _LANES)]
        l1 = lin_v[buf, pl.ds(_LANES, _LANES)]
        acc0 = l0
        acc1 = jnp.where(lane < _F - _LANES, l1, 0.0)
        acc2 = jnp.zeros((_LANES,), jnp.float32)
        acc3 = jnp.zeros((_LANES,), jnp.float32)
        for i in range(_F):
            for j in range(i + 1, _F):
                jc, jo = divmod(j * _D, 128)
                ic, io = divmod(i * _D, 128)
                u0 = a_v[buf, jc * _F + i, pl.ds(jo, _LANES)]
                v0 = a_v[buf, ic * _F + j, pl.ds(io, _LANES)]
                u1 = a_v[buf, jc * _F + i, pl.ds(jo + _LANES, _LANES)]
                v1 = a_v[buf, ic * _F + j, pl.ds(io + _LANES, _LANES)]
                if (i + j) % 2 == 0:
                    acc0 = acc0 + u0 * v0
                    acc1 = acc1 + u1 * v1
                else:
                    acc2 = acc2 + u0 * v0
                    acc3 = acc3 + u1 * v1
        acc_v[pl.ds(lax.rem(s, _LANES) * _LANES, _LANES)] = (
            (acc0 + acc1) + (acc2 + acc3))

    def step(s, buf):
        nbuf = 1 - buf

        @pl.when(s + 1 < bpw)
        def _():
            pltpu.make_async_copy(rows_hbm.at[base], idxv.at[nbuf],
                                  sem_i.at[nbuf]).wait()
            issue_gathers(nbuf)

        wait_gathers(buf)

        @pl.when(s + 2 < bpw)
        def _():
            pltpu.async_copy(rows_hbm.at[base + s + 2], idxv.at[buf],
                             sem_i.at[buf])

        compute(buf, s)

    pltpu.sync_copy(rows_hbm.at[base], idxv.at[0])
    issue_gathers(0)
    pltpu.async_copy(rows_hbm.at[base + 1], idxv.at[1], sem_i.at[1])

    def body(t, carry):
        step(2 * t, 0)
        step(2 * t + 1, 1)

        @pl.when(lax.rem(t, 8) == 7)
        def _():
            total = jnp.zeros((_LANES,), jnp.float32)
            for l in range(_LANES):
                total = total + plsc.load_gather(acc_v, [lane * _LANES + l])
            sig = 1.0 / (1.0 + jnp.exp(-total))
            out_v[pl.ds(2 * t - 14, _LANES)] = sig

        return carry

    lax.fori_loop(0, bpw // 2, body, jnp.int32(0))
    pltpu.sync_copy(out_v, out_hbm.at[pl.ds(base, bpw)])


def kernel(x, offsets, lin_table, lin_bias, ffm_tables):
    b, f = x.shape
    fv, v, d = ffm_tables.shape
    assert f == _F and d == _D and b % (2 * _NW) == 0
    bpw = b // _NW

    ffm_vm = jnp.transpose(ffm_tables, (0, 2, 1))
    ngrid = (v + _VB - 1) // _VB
    t2 = pl.pallas_call(
        _tpose_body,
        grid=(ngrid,),
        in_specs=[pl.BlockSpec((_F, _D, _VB), lambda t: (0, 0, t))],
        out_specs=pl.BlockSpec((_W // 128, _VB, 128), lambda t: (0, t, 0)),
        out_shape=jax.ShapeDtypeStruct((_W // 128, v, 128), jnp.float32),
    )(ffm_vm)

    idx = x + offsets[None, :]
    coff = jnp.arange(_W // 128, dtype=jnp.int32) * v
    ffm_ids = (coff[None, :, None] + idx[:, None, :]).reshape(b, -1)
    rows = jnp.concatenate(
        [ffm_ids, jnp.zeros((b, _NROW - ffm_ids.shape[1]), jnp.int32),
         idx, jnp.zeros((b, 2 * _LANES - _F + 8), jnp.int32)], axis=1)

    lin_flat = lin_table[:, 0] + lin_bias[0] / _F

    mesh = plsc.VectorSubcoreMesh(core_axis_name="c", subcore_axis_name="s")
    run = pl.kernel(
        functools.partial(_ffm_body, bpw=bpw),
        out_type=jax.ShapeDtypeStruct((b,), jnp.float32),
        mesh=mesh,
        compiler_params=pltpu.CompilerParams(
            needs_layout_passes=False, use_tc_tiling_on_sc=False),
        scratch_types=[
            pltpu.VMEM((2, _NROW + 2 * _LANES + 8), jnp.int32),
            pltpu.VMEM((2, _NROW, 128), jnp.float32),
            pltpu.VMEM((2, 2 * _LANES), jnp.float32),
            pltpu.VMEM((bpw,), jnp.float32),
            pltpu.VMEM((_LANES * _LANES,), jnp.float32),
            pltpu.SemaphoreType.DMA((2,)),
            pltpu.SemaphoreType.DMA((2,)),
        ],
    )
    return run(rows, lin_flat, t2.reshape((_W // 128) * v, 128))
